# Initial kernel scaffold; baseline (speedup 1.0000x reference)
import jax
import jax.numpy as jnp
from jax.experimental import pallas as pl


def kernel(x, edge_index, edge_weight, W0, b0, W1, b1):
    raise NotImplementedError("write your pallas kernel here")


# speedup vs baseline: 11.0871x; 11.0871x over previous
import dataclasses
import functools

import jax
import jax.numpy as jnp
from jax import lax
from jax.experimental import pallas as pl
from jax.experimental.pallas import tpu as pltpu
from jax.experimental.pallas import tpu_sc as plsc

_NC = 2
_NS = 16
_L = 16
_NW = _NC * _NS

_GDN = lax.GatherDimensionNumbers(
    offset_dims=(), collapsed_slice_dims=(0,), start_index_map=(0,))


def _splat(vec, j):
    return lax.gather(vec, jnp.full((_L, 1), j, jnp.int32), _GDN, (1,),
                      mode=lax.GatherScatterMode.PROMISE_IN_BOUNDS)


def _vmesh():
    return plsc.VectorSubcoreMesh(core_axis_name="c", subcore_axis_name="s")


def _sc_params():
    cp = pltpu.CompilerParams()
    if "needs_layout_passes" in pltpu.CompilerParams.__dataclass_fields__:
        cp = dataclasses.replace(cp, needs_layout_passes=False)
    return cp


def _deg_partials(col, ew, n_nodes):
    e = col.shape[0]
    ept = e // _NW
    assert ept * _NW == e and ept % 8 == 0

    @functools.partial(
        pl.kernel,
        out_type=jax.ShapeDtypeStruct((_NW, n_nodes), jnp.float32),
        mesh=_vmesh(),
        compiler_params=_sc_params(),
        scratch_types=[
            pltpu.VMEM((ept,), jnp.int32),
            pltpu.VMEM((ept,), jnp.float32),
            pltpu.VMEM((n_nodes,), jnp.float32),
        ],
    )
    def k(col_hbm, ew_hbm, out_hbm, colv, ewv, degv):
        wid = lax.axis_index("s") * _NC + lax.axis_index("c")
        base = wid * ept
        zeros = jnp.zeros((_L,), jnp.float32)

        @pl.loop(0, n_nodes // _L)
        def _(i):
            degv[pl.ds(i * _L, _L)] = zeros

        pltpu.sync_copy(col_hbm.at[pl.ds(base, ept)], colv)
        pltpu.sync_copy(ew_hbm.at[pl.ds(base, ept)], ewv)

        @pl.loop(0, ept // _L)
        def _(j):
            idx = colv[pl.ds(j * _L, _L)]
            w = ewv[pl.ds(j * _L, _L)]
            plsc.addupdate_scatter(degv, [idx], w)

        pltpu.sync_copy(degv, out_hbm.at[wid])

    return k(col, ew)


def _dinv_from_partials(parts):
    n = parts.shape[1]

    def body(p_ref, o_ref):
        d = jnp.sum(p_ref[...], axis=0) + 1.0
        o_ref[...] = lax.rsqrt(d)[:, None]

    return pl.pallas_call(
        body,
        out_shape=jax.ShapeDtypeStruct((n, 1), jnp.float32),
    )(parts)


def _agg_partials(v, row, col, ew, dinv, zeros_slab):
    n, d = v.shape
    e = row.shape[0]
    ept = e // _NW
    ch = 80
    nch = ept // ch
    assert nch * ch == ept
    npt = (-(-n // _NS) + 7) // 8 * 8
    last = n - (_NS - 1) * npt
    assert last > 0 and last % 8 == 0
    npad = _NS * npt

    @functools.partial(
        pl.kernel,
        out_type=jax.ShapeDtypeStruct((_NC, n, d), jnp.float32),
        mesh=_vmesh(),
        compiler_params=_sc_params(),
        scratch_types=[
            pltpu.VMEM((n,), jnp.float32),
            pltpu.VMEM((ch,), jnp.int32),
            pltpu.VMEM((ch,), jnp.int32),
            pltpu.VMEM((ch,), jnp.float32),
            pltpu.VMEM((ch, d), jnp.float32),
            pltpu.VMEM_SHARED((npad, d), jnp.float32),
            pltpu.SemaphoreType.DMA,
        ],
    )
    def k(v_hbm, row_hbm, col_hbm, ew_hbm, dinv_hbm, z_hbm, out_hbm,
          dinvv, rowv, colv, ewv, rows, aggs, sem):
        cid = lax.axis_index("c")
        sid = lax.axis_index("s")
        wid = sid * _NC + cid

        pltpu.sync_copy(dinv_hbm.at[pl.ds(0, n)], dinvv)
        nbase = sid * npt
        pltpu.sync_copy(z_hbm.at[pl.ds(nbase, last)],
                        aggs.at[pl.ds(nbase, last)])

        @pl.when(sid < _NS - 1)
        def _():
            pltpu.sync_copy(z_hbm.at[pl.ds(nbase + last, npt - last)],
                            aggs.at[pl.ds(nbase + last, npt - last)])

        plsc.subcore_barrier()

        @pl.loop(0, nch)
        def _(c):
            base = wid * ept + c * ch
            pltpu.sync_copy(row_hbm.at[pl.ds(base, ch)], rowv)
            pltpu.sync_copy(col_hbm.at[pl.ds(base, ch)], colv)
            pltpu.sync_copy(ew_hbm.at[pl.ds(base, ch)], ewv)
            pltpu.async_copy(v_hbm.at[rowv], rows, sem).wait()
            for g in range(ch // _L):
                sl = pl.ds(g * _L, _L)
                dr = plsc.load_gather(dinvv, [rowv[sl]])
                dc = plsc.load_gather(dinvv, [colv[sl]])
                nrm = dr * ewv[sl] * dc
                for j in range(_L):
                    s = _splat(nrm, j)
                    r = g * _L + j
                    for q in range(d // _L):
                        slq = pl.ds(q * _L, _L)
                        rows[r, slq] = rows[r, slq] * s
            pltpu.sync_copy(rows, aggs.at[colv], add=True)

        plsc.subcore_barrier()
        pltpu.sync_copy(aggs.at[pl.ds(nbase, last)],
                        out_hbm.at[cid].at[pl.ds(nbase, last)])

        @pl.when(sid < _NS - 1)
        def _():
            pltpu.sync_copy(aggs.at[pl.ds(nbase + last, npt - last)],
                            out_hbm.at[cid].at[pl.ds(nbase + last, npt - last)])

    return k(v, row, col, ew, dinv, zeros_slab)


def _dense_layer(parts, v, dinv, w, b):
    n, d = v.shape
    bn = 1000
    grid = n // bn

    def body(p_ref, v_ref, di_ref, w_ref, b_ref, o_ref):
        di = di_ref[...]
        agg = p_ref[0] + p_ref[1] + (di * di) * v_ref[...]
        acc = jnp.dot(agg, w_ref[...],
                      preferred_element_type=jnp.float32,
                      precision=lax.Precision.HIGHEST)
        o_ref[...] = jnp.maximum(acc + b_ref[...], 0.0)

    return pl.pallas_call(
        body,
        grid=(grid,),
        in_specs=[
            pl.BlockSpec((2, bn, d), lambda i: (0, i, 0)),
            pl.BlockSpec((bn, d), lambda i: (i, 0)),
            pl.BlockSpec((bn, 1), lambda i: (i, 0)),
            pl.BlockSpec((d, d), lambda i: (0, 0)),
            pl.BlockSpec((1, d), lambda i: (0, 0)),
        ],
        out_specs=pl.BlockSpec((bn, d), lambda i: (i, 0)),
        out_shape=jax.ShapeDtypeStruct((n, d), jnp.float32),
    )(parts, v, dinv, w, b)


def kernel(x, edge_index, edge_weight, W0, b0, W1, b1):
    n, d = x.shape
    row = edge_index[0]
    col = edge_index[1]
    ew = edge_weight.astype(jnp.float32)
    zeros_slab = jnp.zeros((n, d), jnp.float32)

    deg_parts = _deg_partials(col, ew, n)
    dinv = _dinv_from_partials(deg_parts)
    dinv_flat = dinv[:, 0]

    b0r = b0.reshape(1, d)
    b1r = b1.reshape(1, d)

    p = _agg_partials(x, row, col, ew, dinv_flat, zeros_slab)
    h = _dense_layer(p, x, dinv, W0, b0r)
    p2 = _agg_partials(h, row, col, ew, dinv_flat, zeros_slab)
    out = _dense_layer(p2, h, dinv, W1, b1r)
    return out


# speedup vs baseline: 20.7348x; 1.8702x over previous
import dataclasses
import functools

import jax
import jax.numpy as jnp
from jax import lax
from jax.experimental import pallas as pl
from jax.experimental.pallas import tpu as pltpu
from jax.experimental.pallas import tpu_sc as plsc

_NC = 2
_NS = 16
_L = 16
_NW = _NC * _NS

_GDN = lax.GatherDimensionNumbers(
    offset_dims=(), collapsed_slice_dims=(0,), start_index_map=(0,))


def _splat(vec, j):
    return lax.gather(vec, jnp.full((_L, 1), j, jnp.int32), _GDN, (1,),
                      mode=lax.GatherScatterMode.PROMISE_IN_BOUNDS)


def _vmesh():
    return plsc.VectorSubcoreMesh(core_axis_name="c", subcore_axis_name="s")


def _sc_params():
    cp = pltpu.CompilerParams()
    if "needs_layout_passes" in pltpu.CompilerParams.__dataclass_fields__:
        cp = dataclasses.replace(cp, needs_layout_passes=False)
    return cp


def _deg_partials(col, ew, n_nodes):
    e = col.shape[0]
    ept = e // _NW
    assert ept * _NW == e and ept % 8 == 0

    @functools.partial(
        pl.kernel,
        out_type=jax.ShapeDtypeStruct((_NW, n_nodes), jnp.float32),
        mesh=_vmesh(),
        compiler_params=_sc_params(),
        scratch_types=[
            pltpu.VMEM((ept,), jnp.int32),
            pltpu.VMEM((ept,), jnp.float32),
            pltpu.VMEM((n_nodes,), jnp.float32),
        ],
    )
    def k(col_hbm, ew_hbm, out_hbm, colv, ewv, degv):
        wid = lax.axis_index("s") * _NC + lax.axis_index("c")
        base = wid * ept
        zeros = jnp.zeros((_L,), jnp.float32)

        @pl.loop(0, n_nodes // _L)
        def _(i):
            degv[pl.ds(i * _L, _L)] = zeros

        pltpu.sync_copy(col_hbm.at[pl.ds(base, ept)], colv)
        pltpu.sync_copy(ew_hbm.at[pl.ds(base, ept)], ewv)

        @pl.loop(0, ept // _L)
        def _(j):
            idx = colv[pl.ds(j * _L, _L)]
            w = ewv[pl.ds(j * _L, _L)]
            plsc.addupdate_scatter(degv, [idx], w)

        pltpu.sync_copy(degv, out_hbm.at[wid])

    return k(col, ew)


def _dinv_from_partials(parts):
    n = parts.shape[1]

    def body(p_ref, o_ref):
        d = jnp.sum(p_ref[...], axis=0) + 1.0
        o_ref[...] = lax.rsqrt(d)[:, None]

    return pl.pallas_call(
        body,
        out_shape=jax.ShapeDtypeStruct((n, 1), jnp.float32),
    )(parts)


def _agg_partials(v, pk, dinv, zeros_slab):
    n, d = v.shape
    nw, nch, _, ch = pk.shape
    assert nw == _NW and ch == 80
    nbuf = 3
    npt = (-(-n // _NS) + 7) // 8 * 8
    last = n - (_NS - 1) * npt
    assert last > 0 and last % 8 == 0
    npad = _NS * npt

    @functools.partial(
        pl.kernel,
        out_type=jax.ShapeDtypeStruct((_NC, n, d), jnp.float32),
        mesh=_vmesh(),
        compiler_params=_sc_params(),
        scratch_types=[
            pltpu.VMEM((n,), jnp.float32),
            pltpu.VMEM((3, ch), jnp.int32),
            pltpu.VMEM((3, ch), jnp.int32),
            pltpu.VMEM((3, ch), jnp.int32),
            pltpu.VMEM((ch, d), jnp.float32),
            pltpu.VMEM((ch, d), jnp.float32),
            pltpu.VMEM((ch, d), jnp.float32),
            pltpu.VMEM_SHARED((npad, d), jnp.float32),
            pltpu.SemaphoreType.DMA,
            pltpu.SemaphoreType.DMA,
            pltpu.SemaphoreType.DMA,
            pltpu.SemaphoreType.DMA,
            pltpu.SemaphoreType.DMA,
            pltpu.SemaphoreType.DMA,
        ],
    )
    def k(v_hbm, pk_hbm, dinv_hbm, z_hbm, out_hbm,
          dinvv, i0_, i1_, i2_, b0_, b1_, b2_, aggs,
          g0, g1, g2, s0, s1, s2):
        cid = lax.axis_index("c")
        sid = lax.axis_index("s")
        wid = sid * _NC + cid
        ibufs = (i0_, i1_, i2_)
        bufs = (b0_, b1_, b2_)
        gsems = (g0, g1, g2)
        ssems = (s0, s1, s2)

        pltpu.sync_copy(dinv_hbm.at[pl.ds(0, n)], dinvv)
        nbase = sid * npt
        pltpu.sync_copy(z_hbm.at[pl.ds(nbase, last)],
                        aggs.at[pl.ds(nbase, last)])

        @pl.when(sid < _NS - 1)
        def _():
            pltpu.sync_copy(z_hbm.at[pl.ds(nbase + last, npt - last)],
                            aggs.at[pl.ds(nbase + last, npt - last)])

        plsc.subcore_barrier()

        def i_copy(c, b):
            pltpu.sync_copy(pk_hbm.at[wid].at[c], ibufs[b])

        def g_start(c, b):
            pltpu.make_async_copy(v_hbm.at[ibufs[b].at[0]], bufs[b],
                                  gsems[b]).start()

        def g_wait(c, b):
            pltpu.make_async_copy(v_hbm.at[ibufs[b].at[0]], bufs[b],
                                  gsems[b]).wait()

        def s_start(c, b):
            pltpu.async_copy(bufs[b], aggs.at[ibufs[b].at[1]], ssems[b],
                             add=True)

        def s_wait(c, b):
            pltpu.make_async_copy(bufs[b], aggs.at[ibufs[b].at[1]],
                                  ssems[b]).wait()

        def scale(c, b):
            buf = bufs[b]
            ib = ibufs[b]

            @pl.loop(0, ch // _L)
            def _(g):
                sl = pl.ds(g * _L, _L)
                dr = plsc.load_gather(dinvv, [ib[0, sl]])
                dc = plsc.load_gather(dinvv, [ib[1, sl]])
                ewv = plsc.bitcast(ib[2, sl], jnp.float32)
                nrm = dr * ewv * dc
                for j in range(_L):
                    s = _splat(nrm, j)
                    r = g * _L + j
                    for q in range(d // _L):
                        slq = pl.ds(q * _L, _L)
                        buf[r, slq] = buf[r, slq] * s

        i_copy(0, 0)
        g_start(0, 0)
        i_copy(1, 1)
        g_start(1, 1)
        nmain = (nch // nbuf) * nbuf

        @pl.loop(0, nmain // nbuf)
        def _(i):
            for kpos in range(nbuf):
                c = i * nbuf + kpos

                @pl.when(c >= 1)
                def _():
                    s_wait(c - 1, (kpos - 1) % nbuf)

                @pl.when(c + 2 < nch)
                def _():
                    i_copy(c + 2, (kpos + 2) % nbuf)
                    g_start(c + 2, (kpos + 2) % nbuf)

                g_wait(c, kpos)
                scale(c, kpos)
                s_start(c, kpos)

        for c in range(nmain, nch):
            b = c % nbuf
            s_wait(c - 1, (c - 1) % nbuf)
            g_wait(c, b)
            scale(c, b)
            s_start(c, b)
        s_wait(nch - 1, (nch - 1) % nbuf)

        plsc.subcore_barrier()
        pltpu.sync_copy(aggs.at[pl.ds(nbase, last)],
                        out_hbm.at[cid].at[pl.ds(nbase, last)])

        @pl.when(sid < _NS - 1)
        def _():
            pltpu.sync_copy(aggs.at[pl.ds(nbase + last, npt - last)],
                            out_hbm.at[cid].at[pl.ds(nbase + last, npt - last)])

    return k(v, pk, dinv, zeros_slab)


def _dense_layer(parts, v, dinv, w, b):
    n, d = v.shape
    bn = 1000
    grid = n // bn

    def body(p_ref, v_ref, di_ref, w_ref, b_ref, o_ref):
        di = di_ref[...]
        agg = p_ref[0] + p_ref[1] + (di * di) * v_ref[...]
        acc = jnp.dot(agg, w_ref[...],
                      preferred_element_type=jnp.float32,
                      precision=lax.Precision.HIGHEST)
        o_ref[...] = jnp.maximum(acc + b_ref[...], 0.0)

    return pl.pallas_call(
        body,
        grid=(grid,),
        in_specs=[
            pl.BlockSpec((2, bn, d), lambda i: (0, i, 0)),
            pl.BlockSpec((bn, d), lambda i: (i, 0)),
            pl.BlockSpec((bn, 1), lambda i: (i, 0)),
            pl.BlockSpec((d, d), lambda i: (0, 0)),
            pl.BlockSpec((1, d), lambda i: (0, 0)),
        ],
        out_specs=pl.BlockSpec((bn, d), lambda i: (i, 0)),
        out_shape=jax.ShapeDtypeStruct((n, d), jnp.float32),
    )(parts, v, dinv, w, b)


def kernel(x, edge_index, edge_weight, W0, b0, W1, b1):
    n, d = x.shape
    row = edge_index[0]
    col = edge_index[1]
    ew = edge_weight.astype(jnp.float32)
    zeros_slab = jnp.zeros((n, d), jnp.float32)
    e = row.shape[0]
    ch = 80
    nch = e // _NW // ch
    pk = jnp.stack([row.reshape(_NW, nch, ch),
                    col.reshape(_NW, nch, ch),
                    lax.bitcast_convert_type(ew, jnp.int32)
                       .reshape(_NW, nch, ch)], axis=2)

    deg_parts = _deg_partials(col, ew, n)
    dinv = _dinv_from_partials(deg_parts)
    dinv_flat = dinv[:, 0]

    b0r = b0.reshape(1, d)
    b1r = b1.reshape(1, d)

    p = _agg_partials(x, pk, dinv_flat, zeros_slab)
    h = _dense_layer(p, x, dinv, W0, b0r)
    p2 = _agg_partials(h, pk, dinv_flat, zeros_slab)
    out = _dense_layer(p2, h, dinv, W1, b1r)
    return out


# speedup vs baseline: 26.2150x; 1.2643x over previous
import dataclasses
import functools

import jax
import jax.numpy as jnp
from jax import lax
from jax.experimental import pallas as pl
from jax.experimental.pallas import tpu as pltpu
from jax.experimental.pallas import tpu_sc as plsc

_NC = 2
_NS = 16
_L = 16
_NW = _NC * _NS

_GDN = lax.GatherDimensionNumbers(
    offset_dims=(), collapsed_slice_dims=(0,), start_index_map=(0,))


def _splat(vec, j):
    return lax.gather(vec, jnp.full((_L, 1), j, jnp.int32), _GDN, (1,),
                      mode=lax.GatherScatterMode.PROMISE_IN_BOUNDS)


def _vmesh():
    return plsc.VectorSubcoreMesh(core_axis_name="c", subcore_axis_name="s")


def _sc_params():
    cp = pltpu.CompilerParams()
    if "needs_layout_passes" in pltpu.CompilerParams.__dataclass_fields__:
        cp = dataclasses.replace(cp, needs_layout_passes=False)
    return cp


def _deg_partials(col, ew, n_nodes):
    e = col.shape[0]
    ept = e // _NW
    assert ept * _NW == e and ept % 8 == 0

    @functools.partial(
        pl.kernel,
        out_type=jax.ShapeDtypeStruct((_NW, n_nodes), jnp.float32),
        mesh=_vmesh(),
        compiler_params=_sc_params(),
        scratch_types=[
            pltpu.VMEM((ept,), jnp.int32),
            pltpu.VMEM((ept,), jnp.float32),
            pltpu.VMEM((n_nodes,), jnp.float32),
        ],
    )
    def k(col_hbm, ew_hbm, out_hbm, colv, ewv, degv):
        wid = lax.axis_index("s") * _NC + lax.axis_index("c")
        base = wid * ept
        zeros = jnp.zeros((_L,), jnp.float32)

        @pl.loop(0, n_nodes // _L)
        def _(i):
            degv[pl.ds(i * _L, _L)] = zeros

        pltpu.sync_copy(col_hbm.at[pl.ds(base, ept)], colv)
        pltpu.sync_copy(ew_hbm.at[pl.ds(base, ept)], ewv)

        @pl.loop(0, ept // _L)
        def _(j):
            idx = colv[pl.ds(j * _L, _L)]
            w = ewv[pl.ds(j * _L, _L)]
            plsc.addupdate_scatter(degv, [idx], w)

        pltpu.sync_copy(degv, out_hbm.at[wid])

    return k(col, ew)


def _dinv_from_partials(parts, x):
    n, d = x.shape

    def body(p_ref, x_ref, o_ref, y_ref):
        dg = jnp.sum(p_ref[...], axis=0) + 1.0
        di = lax.rsqrt(dg)[:, None]
        o_ref[...] = di
        y_ref[...] = di * x_ref[...]

    return pl.pallas_call(
        body,
        out_shape=[jax.ShapeDtypeStruct((n, 1), jnp.float32),
                   jax.ShapeDtypeStruct((n, d), jnp.float32)],
    )(parts, x)


def _agg_partials(v, pk, zeros_slab):
    n, d = v.shape
    nw, nch, _, ch = pk.shape
    assert nw == _NW and ch == 80
    nbuf = 4
    npt = (-(-n // _NS) + 7) // 8 * 8
    last = n - (_NS - 1) * npt
    assert last > 0 and last % 8 == 0
    npad = _NS * npt

    @functools.partial(
        pl.kernel,
        out_type=jax.ShapeDtypeStruct((_NC, n, d), jnp.float32),
        mesh=_vmesh(),
        compiler_params=_sc_params(),
        scratch_types=[
            pltpu.VMEM((3, ch), jnp.int32),
            pltpu.VMEM((3, ch), jnp.int32),
            pltpu.VMEM((3, ch), jnp.int32),
            pltpu.VMEM((3, ch), jnp.int32),
            pltpu.VMEM((ch, d), jnp.float32),
            pltpu.VMEM((ch, d), jnp.float32),
            pltpu.VMEM((ch, d), jnp.float32),
            pltpu.VMEM((ch, d), jnp.float32),
            pltpu.VMEM_SHARED((npad, d), jnp.float32),
            pltpu.SemaphoreType.DMA,
            pltpu.SemaphoreType.DMA,
            pltpu.SemaphoreType.DMA,
            pltpu.SemaphoreType.DMA,
            pltpu.SemaphoreType.DMA,
            pltpu.SemaphoreType.DMA,
            pltpu.SemaphoreType.DMA,
            pltpu.SemaphoreType.DMA,
        ],
    )
    def k(v_hbm, pk_hbm, z_hbm, out_hbm,
          i0_, i1_, i2_, i3_, b0_, b1_, b2_, b3_, aggs,
          g0, g1, g2, g3, s0, s1, s2, s3):
        cid = lax.axis_index("c")
        sid = lax.axis_index("s")
        wid = sid * _NC + cid
        ibufs = (i0_, i1_, i2_, i3_)
        bufs = (b0_, b1_, b2_, b3_)
        gsems = (g0, g1, g2, g3)
        ssems = (s0, s1, s2, s3)

        nbase = sid * npt
        pltpu.sync_copy(z_hbm.at[pl.ds(nbase, last)],
                        aggs.at[pl.ds(nbase, last)])

        @pl.when(sid < _NS - 1)
        def _():
            pltpu.sync_copy(z_hbm.at[pl.ds(nbase + last, npt - last)],
                            aggs.at[pl.ds(nbase + last, npt - last)])

        plsc.subcore_barrier()

        def i_copy(c, b):
            pltpu.sync_copy(pk_hbm.at[wid].at[c], ibufs[b])

        def g_start(c, b):
            pltpu.make_async_copy(v_hbm.at[ibufs[b].at[0]], bufs[b],
                                  gsems[b]).start()

        def g_wait(c, b):
            pltpu.make_async_copy(v_hbm.at[ibufs[b].at[0]], bufs[b],
                                  gsems[b]).wait()

        def s_start(c, b):
            pltpu.async_copy(bufs[b], aggs.at[ibufs[b].at[1]], ssems[b],
                             add=True)

        def s_wait(c, b):
            pltpu.make_async_copy(bufs[b], aggs.at[ibufs[b].at[1]],
                                  ssems[b]).wait()

        def scale(c, b):
            buf = bufs[b]
            ib = ibufs[b]

            @pl.loop(0, ch // _L)
            def _(g):
                sl = pl.ds(g * _L, _L)
                nrm = plsc.bitcast(ib[2, sl], jnp.float32)
                for j in range(_L):
                    s = _splat(nrm, j)
                    r = g * _L + j
                    for q in range(d // _L):
                        slq = pl.ds(q * _L, _L)
                        buf[r, slq] = buf[r, slq] * s

        i_copy(0, 0)
        g_start(0, 0)
        i_copy(1, 1)
        g_start(1, 1)
        nmain = (nch // nbuf) * nbuf

        @pl.loop(0, nmain // nbuf)
        def _(i):
            for kpos in range(nbuf):
                c = i * nbuf + kpos

                @pl.when(c >= 2)
                def _():
                    s_wait(c - 2, (kpos - 2) % nbuf)

                @pl.when(c + 2 < nch)
                def _():
                    i_copy(c + 2, (kpos + 2) % nbuf)
                    g_start(c + 2, (kpos + 2) % nbuf)

                g_wait(c, kpos)
                scale(c, kpos)
                s_start(c, kpos)

        for c in range(nmain, nch):
            b = c % nbuf
            s_wait(c - 2, (c - 2) % nbuf)
            g_wait(c, b)
            scale(c, b)
            s_start(c, b)
        for c in range(nch - 2, nch):
            s_wait(c, c % nbuf)

        plsc.subcore_barrier()
        pltpu.sync_copy(aggs.at[pl.ds(nbase, last)],
                        out_hbm.at[cid].at[pl.ds(nbase, last)])

        @pl.when(sid < _NS - 1)
        def _():
            pltpu.sync_copy(aggs.at[pl.ds(nbase + last, npt - last)],
                            out_hbm.at[cid].at[pl.ds(nbase + last, npt - last)])

    return k(v, pk, zeros_slab)


def _dense_layer(parts, y, dinv, w, b):
    n, d = y.shape
    bn = 1000
    grid = n // bn

    def body(p_ref, y_ref, di_ref, w_ref, b_ref, o_ref, ynext_ref):
        di = di_ref[...]
        agg = di * (p_ref[0] + p_ref[1] + y_ref[...])
        acc = jnp.dot(agg, w_ref[...],
                      preferred_element_type=jnp.float32,
                      precision=lax.Precision.HIGHEST)
        h = jnp.maximum(acc + b_ref[...], 0.0)
        o_ref[...] = h
        ynext_ref[...] = di * h

    return pl.pallas_call(
        body,
        grid=(grid,),
        in_specs=[
            pl.BlockSpec((2, bn, d), lambda i: (0, i, 0)),
            pl.BlockSpec((bn, d), lambda i: (i, 0)),
            pl.BlockSpec((bn, 1), lambda i: (i, 0)),
            pl.BlockSpec((d, d), lambda i: (0, 0)),
            pl.BlockSpec((1, d), lambda i: (0, 0)),
        ],
        out_specs=[pl.BlockSpec((bn, d), lambda i: (i, 0)),
                   pl.BlockSpec((bn, d), lambda i: (i, 0))],
        out_shape=[jax.ShapeDtypeStruct((n, d), jnp.float32),
                   jax.ShapeDtypeStruct((n, d), jnp.float32)],
    )(parts, y, dinv, w, b)


def kernel(x, edge_index, edge_weight, W0, b0, W1, b1):
    n, d = x.shape
    row = edge_index[0]
    col = edge_index[1]
    ew = edge_weight.astype(jnp.float32)
    zeros_slab = jnp.zeros((n, d), jnp.float32)
    e = row.shape[0]
    ch = 80
    nch = e // _NW // ch
    pk = jnp.stack([row.reshape(_NW, nch, ch),
                    col.reshape(_NW, nch, ch),
                    lax.bitcast_convert_type(ew, jnp.int32)
                       .reshape(_NW, nch, ch)], axis=2)

    deg_parts = _deg_partials(col, ew, n)
    dinv, y1 = _dinv_from_partials(deg_parts, x)

    b0r = b0.reshape(1, d)
    b1r = b1.reshape(1, d)

    p = _agg_partials(y1, pk, zeros_slab)
    h, y2 = _dense_layer(p, y1, dinv, W0, b0r)
    p2 = _agg_partials(y2, pk, zeros_slab)
    out, _ = _dense_layer(p2, y2, dinv, W1, b1r)
    return out


# speedup vs baseline: 28.9939x; 1.1060x over previous
import dataclasses
import functools

import jax
import jax.numpy as jnp
from jax import lax
from jax.experimental import pallas as pl
from jax.experimental.pallas import tpu as pltpu
from jax.experimental.pallas import tpu_sc as plsc

_NC = 2
_NS = 16
_L = 16
_NW = _NC * _NS

_GDN = lax.GatherDimensionNumbers(
    offset_dims=(), collapsed_slice_dims=(0,), start_index_map=(0,))


def _splat(vec, j):
    return lax.gather(vec, jnp.full((_L, 1), j, jnp.int32), _GDN, (1,),
                      mode=lax.GatherScatterMode.PROMISE_IN_BOUNDS)


def _vmesh():
    return plsc.VectorSubcoreMesh(core_axis_name="c", subcore_axis_name="s")


def _sc_params():
    cp = pltpu.CompilerParams()
    if "needs_layout_passes" in pltpu.CompilerParams.__dataclass_fields__:
        cp = dataclasses.replace(cp, needs_layout_passes=False)
    return cp


def _deg_partials(col, ew, n_nodes):
    e = col.shape[0]
    ept = e // _NW
    assert ept * _NW == e and ept % 8 == 0

    @functools.partial(
        pl.kernel,
        out_type=jax.ShapeDtypeStruct((_NW, n_nodes), jnp.float32),
        mesh=_vmesh(),
        compiler_params=_sc_params(),
        scratch_types=[
            pltpu.VMEM((ept,), jnp.int32),
            pltpu.VMEM((ept,), jnp.float32),
            pltpu.VMEM((n_nodes,), jnp.float32),
        ],
    )
    def k(col_hbm, ew_hbm, out_hbm, colv, ewv, degv):
        wid = lax.axis_index("s") * _NC + lax.axis_index("c")
        base = wid * ept
        zeros = jnp.zeros((_L,), jnp.float32)

        @pl.loop(0, n_nodes // _L)
        def _(i):
            degv[pl.ds(i * _L, _L)] = zeros

        pltpu.sync_copy(col_hbm.at[pl.ds(base, ept)], colv)
        pltpu.sync_copy(ew_hbm.at[pl.ds(base, ept)], ewv)

        @pl.loop(0, ept // _L)
        def _(j):
            idx = colv[pl.ds(j * _L, _L)]
            w = ewv[pl.ds(j * _L, _L)]
            plsc.addupdate_scatter(degv, [idx], w)

        pltpu.sync_copy(degv, out_hbm.at[wid])

    return k(col, ew)


def _dinv_from_partials(parts, x):
    n, d = x.shape

    def body(p_ref, x_ref, o_ref, y_ref):
        dg = jnp.sum(p_ref[...], axis=0) + 1.0
        di = lax.rsqrt(dg)[:, None]
        o_ref[...] = di
        y_ref[...] = di * x_ref[...]

    return pl.pallas_call(
        body,
        out_shape=[jax.ShapeDtypeStruct((n, 1), jnp.float32),
                   jax.ShapeDtypeStruct((n, d), jnp.float32)],
    )(parts, x)


def _agg_partials(v, pk, zeros_slab):
    n, d = v.shape
    nw, nch, _, ch = pk.shape
    assert nw == _NW and ch == 80
    nbuf = 4
    npt = (-(-n // _NS) + 7) // 8 * 8
    last = n - (_NS - 1) * npt
    assert last > 0 and last % 8 == 0
    npad = _NS * npt

    @functools.partial(
        pl.kernel,
        out_type=jax.ShapeDtypeStruct((_NC, n, d), jnp.float32),
        mesh=_vmesh(),
        compiler_params=_sc_params(),
        scratch_types=[
            pltpu.VMEM((3, ch), jnp.int32),
            pltpu.VMEM((3, ch), jnp.int32),
            pltpu.VMEM((3, ch), jnp.int32),
            pltpu.VMEM((3, ch), jnp.int32),
            pltpu.VMEM((ch, d), jnp.float32),
            pltpu.VMEM((ch, d), jnp.float32),
            pltpu.VMEM((ch, d), jnp.float32),
            pltpu.VMEM((ch, d), jnp.float32),
            pltpu.VMEM_SHARED((npad, d), jnp.float32),
            pltpu.SemaphoreType.DMA,
            pltpu.SemaphoreType.DMA,
            pltpu.SemaphoreType.DMA,
            pltpu.SemaphoreType.DMA,
            pltpu.SemaphoreType.DMA,
            pltpu.SemaphoreType.DMA,
            pltpu.SemaphoreType.DMA,
            pltpu.SemaphoreType.DMA,
            pltpu.SemaphoreType.DMA,
            pltpu.SemaphoreType.DMA,
            pltpu.SemaphoreType.DMA,
            pltpu.SemaphoreType.DMA,
        ],
    )
    def k(v_hbm, pk_hbm, z_hbm, out_hbm,
          i0_, i1_, i2_, i3_, b0_, b1_, b2_, b3_, aggs,
          g0, g1, g2, g3, s0, s1, s2, s3, q0, q1, q2, q3):
        cid = lax.axis_index("c")
        sid = lax.axis_index("s")
        wid = sid * _NC + cid
        ibufs = (i0_, i1_, i2_, i3_)
        bufs = (b0_, b1_, b2_, b3_)
        gsems = (g0, g1, g2, g3)
        ssems = (s0, s1, s2, s3)
        isems = (q0, q1, q2, q3)

        nbase = sid * npt
        pltpu.sync_copy(z_hbm.at[pl.ds(nbase, last)],
                        aggs.at[pl.ds(nbase, last)])

        @pl.when(sid < _NS - 1)
        def _():
            pltpu.sync_copy(z_hbm.at[pl.ds(nbase + last, npt - last)],
                            aggs.at[pl.ds(nbase + last, npt - last)])

        plsc.subcore_barrier()

        def i_start(c, b):
            pltpu.make_async_copy(pk_hbm.at[wid].at[c], ibufs[b],
                                  isems[b]).start()

        def i_wait(c, b):
            pltpu.make_async_copy(pk_hbm.at[wid].at[c], ibufs[b],
                                  isems[b]).wait()

        def g_start(c, b):
            pltpu.make_async_copy(v_hbm.at[ibufs[b].at[0]], bufs[b],
                                  gsems[b]).start()

        def g_wait(c, b):
            pltpu.make_async_copy(v_hbm.at[ibufs[b].at[0]], bufs[b],
                                  gsems[b]).wait()

        def s_start(c, b):
            pltpu.async_copy(bufs[b], aggs.at[ibufs[b].at[1]], ssems[b],
                             add=True)

        def s_wait(c, b):
            pltpu.make_async_copy(bufs[b], aggs.at[ibufs[b].at[1]],
                                  ssems[b]).wait()

        def scale(c, b):
            buf = bufs[b]
            ib = ibufs[b]

            @pl.loop(0, ch // _L)
            def _(g):
                sl = pl.ds(g * _L, _L)
                nrm = plsc.bitcast(ib[2, sl], jnp.float32)
                for j in range(_L):
                    s = _splat(nrm, j)
                    r = g * _L + j
                    for q in range(d // _L):
                        slq = pl.ds(q * _L, _L)
                        buf[r, slq] = buf[r, slq] * s

        i_start(0, 0)
        i_start(1, 1)
        i_wait(0, 0)
        g_start(0, 0)
        i_wait(1, 1)
        g_start(1, 1)
        nmain = (nch // nbuf) * nbuf

        @pl.loop(0, nmain // nbuf)
        def _(i):
            for kpos in range(nbuf):
                c = i * nbuf + kpos

                @pl.when(c >= 2)
                def _():
                    s_wait(c - 2, (kpos - 2) % nbuf)

                @pl.when(c + 2 < nch)
                def _():
                    i_start(c + 2, (kpos + 2) % nbuf)

                g_wait(c, kpos)
                scale(c, kpos)

                @pl.when(c + 2 < nch)
                def _():
                    i_wait(c + 2, (kpos + 2) % nbuf)
                    g_start(c + 2, (kpos + 2) % nbuf)

                s_start(c, kpos)

        for c in range(nmain, nch):
            b = c % nbuf
            s_wait(c - 2, (c - 2) % nbuf)
            g_wait(c, b)
            scale(c, b)
            s_start(c, b)
        for c in range(nch - 2, nch):
            s_wait(c, c % nbuf)

        plsc.subcore_barrier()
        pltpu.sync_copy(aggs.at[pl.ds(nbase, last)],
                        out_hbm.at[cid].at[pl.ds(nbase, last)])

        @pl.when(sid < _NS - 1)
        def _():
            pltpu.sync_copy(aggs.at[pl.ds(nbase + last, npt - last)],
                            out_hbm.at[cid].at[pl.ds(nbase + last, npt - last)])

    return k(v, pk, zeros_slab)


def _dense_layer(parts, y, dinv, w, b):
    n, d = y.shape
    bn = 1000
    grid = n // bn

    def body(p_ref, y_ref, di_ref, w_ref, b_ref, o_ref, ynext_ref):
        di = di_ref[...]
        agg = di * (p_ref[0] + p_ref[1] + y_ref[...])
        acc = jnp.dot(agg, w_ref[...],
                      preferred_element_type=jnp.float32,
                      precision=lax.Precision.HIGHEST)
        h = jnp.maximum(acc + b_ref[...], 0.0)
        o_ref[...] = h
        ynext_ref[...] = di * h

    return pl.pallas_call(
        body,
        grid=(grid,),
        in_specs=[
            pl.BlockSpec((2, bn, d), lambda i: (0, i, 0)),
            pl.BlockSpec((bn, d), lambda i: (i, 0)),
            pl.BlockSpec((bn, 1), lambda i: (i, 0)),
            pl.BlockSpec((d, d), lambda i: (0, 0)),
            pl.BlockSpec((1, d), lambda i: (0, 0)),
        ],
        out_specs=[pl.BlockSpec((bn, d), lambda i: (i, 0)),
                   pl.BlockSpec((bn, d), lambda i: (i, 0))],
        out_shape=[jax.ShapeDtypeStruct((n, d), jnp.float32),
                   jax.ShapeDtypeStruct((n, d), jnp.float32)],
    )(parts, y, dinv, w, b)


def kernel(x, edge_index, edge_weight, W0, b0, W1, b1):
    n, d = x.shape
    row = edge_index[0]
    col = edge_index[1]
    ew = edge_weight.astype(jnp.float32)
    zeros_slab = jnp.zeros((n, d), jnp.float32)
    e = row.shape[0]
    ch = 80
    nch = e // _NW // ch
    pk = jnp.stack([row.reshape(_NW, nch, ch),
                    col.reshape(_NW, nch, ch),
                    lax.bitcast_convert_type(ew, jnp.int32)
                       .reshape(_NW, nch, ch)], axis=2)

    deg_parts = _deg_partials(col, ew, n)
    dinv, y1 = _dinv_from_partials(deg_parts, x)

    b0r = b0.reshape(1, d)
    b1r = b1.reshape(1, d)

    p = _agg_partials(y1, pk, zeros_slab)
    h, y2 = _dense_layer(p, y1, dinv, W0, b0r)
    p2 = _agg_partials(y2, pk, zeros_slab)
    out, _ = _dense_layer(p2, y2, dinv, W1, b1r)
    return out


# speedup vs baseline: 29.8702x; 1.0302x over previous
import dataclasses
import functools

import jax
import jax.numpy as jnp
from jax import lax
from jax.experimental import pallas as pl
from jax.experimental.pallas import tpu as pltpu
from jax.experimental.pallas import tpu_sc as plsc

_NC = 2
_NS = 16
_L = 16
_NW = _NC * _NS

_GDN = lax.GatherDimensionNumbers(
    offset_dims=(), collapsed_slice_dims=(0,), start_index_map=(0,))


def _splat(vec, j):
    return lax.gather(vec, jnp.full((_L, 1), j, jnp.int32), _GDN, (1,),
                      mode=lax.GatherScatterMode.PROMISE_IN_BOUNDS)


def _vmesh():
    return plsc.VectorSubcoreMesh(core_axis_name="c", subcore_axis_name="s")


def _sc_params():
    cp = pltpu.CompilerParams()
    if "needs_layout_passes" in pltpu.CompilerParams.__dataclass_fields__:
        cp = dataclasses.replace(cp, needs_layout_passes=False)
    return cp


def _deg_partials(col, ew, n_nodes):
    e = col.shape[0]
    ept = e // _NW
    assert ept * _NW == e and ept % 8 == 0

    @functools.partial(
        pl.kernel,
        out_type=jax.ShapeDtypeStruct((_NW, n_nodes), jnp.float32),
        mesh=_vmesh(),
        compiler_params=_sc_params(),
        scratch_types=[
            pltpu.VMEM((ept,), jnp.int32),
            pltpu.VMEM((ept,), jnp.float32),
            pltpu.VMEM((n_nodes,), jnp.float32),
        ],
    )
    def k(col_hbm, ew_hbm, out_hbm, colv, ewv, degv):
        wid = lax.axis_index("s") * _NC + lax.axis_index("c")
        base = wid * ept
        zeros = jnp.zeros((_L,), jnp.float32)

        @pl.loop(0, n_nodes // _L)
        def _(i):
            degv[pl.ds(i * _L, _L)] = zeros

        pltpu.sync_copy(col_hbm.at[pl.ds(base, ept)], colv)
        pltpu.sync_copy(ew_hbm.at[pl.ds(base, ept)], ewv)

        @pl.loop(0, ept // _L)
        def _(j):
            idx = colv[pl.ds(j * _L, _L)]
            w = ewv[pl.ds(j * _L, _L)]
            plsc.addupdate_scatter(degv, [idx], w)

        pltpu.sync_copy(degv, out_hbm.at[wid])

    return k(col, ew)


def _dinv_from_partials(parts, x):
    n, d = x.shape

    def body(p_ref, x_ref, o_ref, y_ref):
        dg = jnp.sum(p_ref[...], axis=0) + 1.0
        di = lax.rsqrt(dg)[:, None]
        o_ref[...] = di
        y_ref[...] = di * x_ref[...]

    return pl.pallas_call(
        body,
        out_shape=[jax.ShapeDtypeStruct((n, 1), jnp.float32),
                   jax.ShapeDtypeStruct((n, d), jnp.float32)],
    )(parts, x)


def _agg_partials(v, pk, zeros_slab):
    n, d = v.shape
    nw, nch, _, ch = pk.shape
    assert nw == _NW and ch == 80
    nbuf = 4
    npt = (-(-n // _NS) + 7) // 8 * 8
    last = n - (_NS - 1) * npt
    assert last > 0 and last % 8 == 0
    npad = _NS * npt

    @functools.partial(
        pl.kernel,
        out_type=jax.ShapeDtypeStruct((_NC, n, d), jnp.float32),
        mesh=_vmesh(),
        compiler_params=_sc_params(),
        scratch_types=[
            pltpu.VMEM((3, ch), jnp.int32),
            pltpu.VMEM((3, ch), jnp.int32),
            pltpu.VMEM((3, ch), jnp.int32),
            pltpu.VMEM((3, ch), jnp.int32),
            pltpu.VMEM((3, ch), jnp.int32),
            pltpu.VMEM((3, ch), jnp.int32),
            pltpu.VMEM((3, ch), jnp.int32),
            pltpu.VMEM((3, ch), jnp.int32),
            pltpu.VMEM((ch, d), jnp.float32),
            pltpu.VMEM((ch, d), jnp.float32),
            pltpu.VMEM((ch, d), jnp.float32),
            pltpu.VMEM((ch, d), jnp.float32),
            pltpu.VMEM_SHARED((npad, d), jnp.float32),
            pltpu.SemaphoreType.DMA,
            pltpu.SemaphoreType.DMA,
            pltpu.SemaphoreType.DMA,
            pltpu.SemaphoreType.DMA,
            pltpu.SemaphoreType.DMA,
            pltpu.SemaphoreType.DMA,
            pltpu.SemaphoreType.DMA,
            pltpu.SemaphoreType.DMA,
            pltpu.SemaphoreType.DMA,
            pltpu.SemaphoreType.DMA,
            pltpu.SemaphoreType.DMA,
            pltpu.SemaphoreType.DMA,
            pltpu.SemaphoreType.DMA,
            pltpu.SemaphoreType.DMA,
            pltpu.SemaphoreType.DMA,
            pltpu.SemaphoreType.DMA,
        ],
    )
    def k(v_hbm, pk_hbm, z_hbm, out_hbm,
          i0_, i1_, i2_, i3_, i4_, i5_, i6_, i7_,
          b0_, b1_, b2_, b3_, aggs,
          g0, g1, g2, g3, s0, s1, s2, s3,
          q0, q1, q2, q3, q4, q5, q6, q7):
        cid = lax.axis_index("c")
        sid = lax.axis_index("s")
        wid = sid * _NC + cid
        ibufs = (i0_, i1_, i2_, i3_, i4_, i5_, i6_, i7_)
        bufs = (b0_, b1_, b2_, b3_)
        gsems = (g0, g1, g2, g3)
        ssems = (s0, s1, s2, s3)
        isems = (q0, q1, q2, q3, q4, q5, q6, q7)

        nbase = sid * npt
        pltpu.sync_copy(z_hbm.at[pl.ds(nbase, last)],
                        aggs.at[pl.ds(nbase, last)])

        @pl.when(sid < _NS - 1)
        def _():
            pltpu.sync_copy(z_hbm.at[pl.ds(nbase + last, npt - last)],
                            aggs.at[pl.ds(nbase + last, npt - last)])

        plsc.subcore_barrier()

        def i_start(c, ib):
            pltpu.make_async_copy(pk_hbm.at[wid].at[c], ibufs[ib],
                                  isems[ib]).start()

        def i_wait(c, ib):
            pltpu.make_async_copy(pk_hbm.at[wid].at[c], ibufs[ib],
                                  isems[ib]).wait()

        def g_start(c, bb, ib):
            pltpu.make_async_copy(v_hbm.at[ibufs[ib].at[0]], bufs[bb],
                                  gsems[bb]).start()

        def g_wait(c, bb, ib):
            pltpu.make_async_copy(v_hbm.at[ibufs[ib].at[0]], bufs[bb],
                                  gsems[bb]).wait()

        def s_start(c, bb, ib):
            pltpu.async_copy(bufs[bb], aggs.at[ibufs[ib].at[1]], ssems[bb],
                             add=True)

        def s_wait(c, bb, ib):
            pltpu.make_async_copy(bufs[bb], aggs.at[ibufs[ib].at[1]],
                                  ssems[bb]).wait()

        def scale(c, bb, ib):
            buf = bufs[bb]
            ib = ibufs[ib]

            @pl.loop(0, ch // _L)
            def _(g):
                sl = pl.ds(g * _L, _L)
                nrm = plsc.bitcast(ib[2, sl], jnp.float32)
                for j in range(_L):
                    s = _splat(nrm, j)
                    r = g * _L + j
                    for q in range(d // _L):
                        slq = pl.ds(q * _L, _L)
                        buf[r, slq] = buf[r, slq] * s

        nib = 8
        for c0 in range(4):
            i_start(c0, c0)
        i_wait(0, 0)
        g_start(0, 0, 0)
        i_wait(1, 1)
        g_start(1, 1, 1)
        nmain = (nch // nib) * nib

        @pl.loop(0, nmain // nib)
        def _(i):
            for kpos in range(nib):
                c = i * nib + kpos
                bb = kpos % nbuf

                @pl.when(c >= 2)
                def _():
                    s_wait(c - 2, (kpos - 2) % nbuf, (kpos - 2) % nib)

                i_start(c + 4, (kpos + 4) % nib)
                i_wait(c + 2, (kpos + 2) % nib)
                g_start(c + 2, (kpos + 2) % nbuf, (kpos + 2) % nib)
                g_wait(c, bb, kpos)
                scale(c, bb, kpos)
                s_start(c, bb, kpos)

        for c in range(nmain, nch):
            s_wait(c - 2, (c - 2) % nbuf, (c - 2) % nib)
            if c + 4 < nch:
                i_start(c + 4, (c + 4) % nib)
            if c + 2 < nch:
                i_wait(c + 2, (c + 2) % nib)
                g_start(c + 2, (c + 2) % nbuf, (c + 2) % nib)
            g_wait(c, c % nbuf, c % nib)
            scale(c, c % nbuf, c % nib)
            s_start(c, c % nbuf, c % nib)
        for c in range(nch - 2, nch):
            s_wait(c, c % nbuf, c % nib)

        plsc.subcore_barrier()
        pltpu.sync_copy(aggs.at[pl.ds(nbase, last)],
                        out_hbm.at[cid].at[pl.ds(nbase, last)])

        @pl.when(sid < _NS - 1)
        def _():
            pltpu.sync_copy(aggs.at[pl.ds(nbase + last, npt - last)],
                            out_hbm.at[cid].at[pl.ds(nbase + last, npt - last)])

    return k(v, pk, zeros_slab)


def _dense_layer(parts, y, dinv, w, b):
    n, d = y.shape
    bn = 1000
    grid = n // bn

    def body(p_ref, y_ref, di_ref, w_ref, b_ref, o_ref, ynext_ref):
        di = di_ref[...]
        agg = di * (p_ref[0] + p_ref[1] + y_ref[...])
        acc = jnp.dot(agg, w_ref[...],
                      preferred_element_type=jnp.float32,
                      precision=lax.Precision.HIGHEST)
        h = jnp.maximum(acc + b_ref[...], 0.0)
        o_ref[...] = h
        ynext_ref[...] = di * h

    return pl.pallas_call(
        body,
        grid=(grid,),
        in_specs=[
            pl.BlockSpec((2, bn, d), lambda i: (0, i, 0)),
            pl.BlockSpec((bn, d), lambda i: (i, 0)),
            pl.BlockSpec((bn, 1), lambda i: (i, 0)),
            pl.BlockSpec((d, d), lambda i: (0, 0)),
            pl.BlockSpec((1, d), lambda i: (0, 0)),
        ],
        out_specs=[pl.BlockSpec((bn, d), lambda i: (i, 0)),
                   pl.BlockSpec((bn, d), lambda i: (i, 0))],
        out_shape=[jax.ShapeDtypeStruct((n, d), jnp.float32),
                   jax.ShapeDtypeStruct((n, d), jnp.float32)],
    )(parts, y, dinv, w, b)


def kernel(x, edge_index, edge_weight, W0, b0, W1, b1):
    n, d = x.shape
    row = edge_index[0]
    col = edge_index[1]
    ew = edge_weight.astype(jnp.float32)
    zeros_slab = jnp.zeros((n, d), jnp.float32)
    e = row.shape[0]
    ch = 80
    nch = e // _NW // ch
    pk = jnp.stack([row.reshape(_NW, nch, ch),
                    col.reshape(_NW, nch, ch),
                    lax.bitcast_convert_type(ew, jnp.int32)
                       .reshape(_NW, nch, ch)], axis=2)

    deg_parts = _deg_partials(col, ew, n)
    dinv, y1 = _dinv_from_partials(deg_parts, x)

    b0r = b0.reshape(1, d)
    b1r = b1.reshape(1, d)

    p = _agg_partials(y1, pk, zeros_slab)
    h, y2 = _dense_layer(p, y1, dinv, W0, b0r)
    p2 = _agg_partials(y2, pk, zeros_slab)
    out, _ = _dense_layer(p2, y2, dinv, W1, b1r)
    return out
